# baseline (speedup 1.0000x reference)
import jax
import jax.numpy as jnp
from jax.experimental import pallas as pl
from jax.experimental.pallas import tpu as pltpu

N = 10000
D = 128
BM = 200
G = (N // 2) // BM


def _fused_gcn_kernel2(a0_ref, a1_ref, ax_ref, wrT_ref, wrb_ref, wT_ref,
                       wb_ref, out_ref, h2_ref):
    @pl.when(pl.program_id(0) == 0)
    def _compute_h2():
        h = jnp.dot(ax_ref[...], wrT_ref[...],
                    preferred_element_type=jnp.float32) + wrb_ref[...]
        h2_ref[...] = jnp.dot(jnp.maximum(h, 0.0), wT_ref[...],
                              preferred_element_type=jnp.float32)

    t0 = jnp.dot(a0_ref[...], h2_ref[...], preferred_element_type=jnp.float32)
    t1 = jnp.dot(a1_ref[...], h2_ref[...], preferred_element_type=jnp.float32)
    out_ref[0] = jnp.maximum(t0 + wb_ref[...], 0.0)
    out_ref[1] = jnp.maximum(t1 + wb_ref[...], 0.0)


@jax.jit
def _run(A, AX, WrT, Wr_b, WT, W_b):
    out = pl.pallas_call(
        _fused_gcn_kernel2,
        grid=(G,),
        in_specs=[
            pl.BlockSpec((BM, N), lambda i: (i, 0)),
            pl.BlockSpec((BM, N), lambda i: (i + G, 0)),
            pl.BlockSpec((N, D), lambda i: (0, 0)),
            pl.BlockSpec((D, D), lambda i: (0, 0)),
            pl.BlockSpec((1, D), lambda i: (0, 0)),
            pl.BlockSpec((D, D), lambda i: (0, 0)),
            pl.BlockSpec((1, D), lambda i: (0, 0)),
        ],
        out_specs=pl.BlockSpec((2, BM, D), lambda i: (0, i, 0)),
        out_shape=jax.ShapeDtypeStruct((2, N // 2, D), jnp.float32),
        scratch_shapes=[pltpu.VMEM((N, D), jnp.float32)],
        compiler_params=pltpu.CompilerParams(
            dimension_semantics=("arbitrary",),
        ),
    )(A, A, AX, WrT, Wr_b, WT, W_b)
    return out.reshape(N, D)[None, :, :]


def kernel(A, AX, Wr_w, Wr_b, W_w, W_b):
    return _run(A, AX, Wr_w.T, Wr_b.reshape(1, D), W_w.T, W_b.reshape(1, D))


# speedup vs baseline: 1.0053x; 1.0053x over previous
import jax
import jax.numpy as jnp
from jax.experimental import pallas as pl
from jax.experimental.pallas import tpu as pltpu

N = 10000
D = 128
BM = 400


def _fused_gcn_kernel(a_ref, ax_ref, wrT_ref, wrb_ref, wT_ref, wb_ref,
                      out_ref, h2_ref):
    @pl.when(pl.program_id(0) == 0)
    def _compute_h2():
        h = jnp.dot(ax_ref[...], wrT_ref[...],
                    preferred_element_type=jnp.float32) + wrb_ref[...]
        h2_ref[...] = jnp.dot(jnp.maximum(h, 0.0), wT_ref[...],
                              preferred_element_type=jnp.float32)

    temp = jnp.dot(a_ref[...], h2_ref[...], preferred_element_type=jnp.float32)
    out_ref[...] = jnp.maximum(temp + wb_ref[...], 0.0)


@jax.jit
def _run(A, AX, WrT, Wr_b, WT, W_b):
    out = pl.pallas_call(
        _fused_gcn_kernel,
        grid=(N // BM,),
        in_specs=[
            pl.BlockSpec((BM, N), lambda i: (i, 0)),
            pl.BlockSpec((N, D), lambda i: (0, 0)),
            pl.BlockSpec((D, D), lambda i: (0, 0)),
            pl.BlockSpec((1, D), lambda i: (0, 0)),
            pl.BlockSpec((D, D), lambda i: (0, 0)),
            pl.BlockSpec((1, D), lambda i: (0, 0)),
        ],
        out_specs=pl.BlockSpec((BM, D), lambda i: (i, 0)),
        out_shape=jax.ShapeDtypeStruct((N, D), jnp.float32),
        scratch_shapes=[pltpu.VMEM((N, D), jnp.float32)],
        compiler_params=pltpu.CompilerParams(
            dimension_semantics=("arbitrary",),
        ),
    )(A, AX, WrT, Wr_b, WT, W_b)
    return out[None, :, :]


def kernel(A, AX, Wr_w, Wr_b, W_w, W_b):
    return _run(A, AX, Wr_w.T, Wr_b.reshape(1, D), W_w.T, W_b.reshape(1, D))


# speedup vs baseline: 1.0334x; 1.0280x over previous
import jax
import jax.numpy as jnp
from jax.experimental import pallas as pl
from jax.experimental.pallas import tpu as pltpu

N = 10000
D = 128
BM = 400


def _dot_t(x, w):
    return jax.lax.dot_general(x, w, (((1,), (1,)), ((), ())),
                               preferred_element_type=jnp.float32)


def _fused_gcn_kernel(a_ref, ax_ref, wr_ref, wrb_ref, w_ref, wb_ref,
                      out_ref, h2_ref):
    @pl.when(pl.program_id(0) == 0)
    def _compute_h2():
        h = _dot_t(ax_ref[...], wr_ref[...]) + wrb_ref[...]
        h2_ref[...] = _dot_t(jnp.maximum(h, 0.0), w_ref[...])

    temp = jnp.dot(a_ref[...], h2_ref[...], preferred_element_type=jnp.float32)
    out_ref[...] = jnp.maximum(temp + wb_ref[...], 0.0)


@jax.jit
def _run(A, AX, Wr, Wr_b, W, W_b):
    out = pl.pallas_call(
        _fused_gcn_kernel,
        grid=(N // BM,),
        in_specs=[
            pl.BlockSpec((BM, N), lambda i: (i, 0)),
            pl.BlockSpec((N, D), lambda i: (0, 0)),
            pl.BlockSpec((D, D), lambda i: (0, 0)),
            pl.BlockSpec((1, D), lambda i: (0, 0)),
            pl.BlockSpec((D, D), lambda i: (0, 0)),
            pl.BlockSpec((1, D), lambda i: (0, 0)),
        ],
        out_specs=pl.BlockSpec((BM, D), lambda i: (i, 0)),
        out_shape=jax.ShapeDtypeStruct((N, D), jnp.float32),
        scratch_shapes=[pltpu.VMEM((N, D), jnp.float32)],
        compiler_params=pltpu.CompilerParams(
            dimension_semantics=("arbitrary",),
        ),
    )(A, AX, Wr, Wr_b, W, W_b)
    return out[None, :, :]


def kernel(A, AX, Wr_w, Wr_b, W_w, W_b):
    return _run(A, AX, Wr_w, Wr_b.reshape(1, D), W_w, W_b.reshape(1, D))
